# Initial kernel scaffold; baseline (speedup 1.0000x reference)
import jax
import jax.numpy as jnp
from jax.experimental import pallas as pl


def kernel(ze, e):
    raise NotImplementedError("write your pallas kernel here")


# speedup vs baseline: 1.6991x; 1.6991x over previous
import jax
import jax.numpy as jnp
from jax import lax
from jax.experimental import pallas as pl

B = 4096
K = 8192
D = 3
BBLK = 64
NBLK = B // BBLK


def _vq_kernel(q_ref, et_ref, z_ref, zq_ref):
    def body(i, _):
        zcols = []
        zqcols = []
        for d in range(D):
            erow = et_ref[d : d + 1, :]
            qcol = q_ref[pl.ds(i * BBLK, BBLK), d : d + 1]
            d2 = (qcol - erow) ** 2
            mind = jnp.min(d2, axis=1, keepdims=True)
            iota = lax.broadcasted_iota(jnp.int32, (BBLK, K), 1)
            zc = jnp.min(
                jnp.where(d2 == mind, iota, K), axis=1, keepdims=True
            )
            zqc = jnp.sum(
                jnp.where(iota == zc, erow, 0.0), axis=1, keepdims=True
            )
            zcols.append(zc)
            zqcols.append(zqc)
        z_ref[pl.ds(i * BBLK, BBLK), :] = jnp.concatenate(zcols, axis=1)
        zq_ref[pl.ds(i * BBLK, BBLK), :] = jnp.concatenate(zqcols, axis=1)
        return 0

    lax.fori_loop(0, NBLK, body, 0)


def kernel(ze, e):
    q = ze.reshape(B, D)
    et = e.T
    z, zq = pl.pallas_call(
        _vq_kernel,
        out_shape=[
            jax.ShapeDtypeStruct((B, D), jnp.int32),
            jax.ShapeDtypeStruct((B, D), jnp.float32),
        ],
    )(q, et)
    return (z, zq)


# speedup vs baseline: 4.5609x; 2.6843x over previous
import functools
import jax
import jax.numpy as jnp
from jax import lax
from jax.experimental import pallas as pl
from jax.experimental.pallas import tpu as pltpu, tpu_sc as plsc

B = 4096
K = 8192
D = 3
NBUCK = 2048
QS = 512
NQV = QS // 16
NKV = K // 16
NBV = NBUCK // 16

_mesh = plsc.VectorSubcoreMesh(core_axis_name="c", subcore_axis_name="s")


@functools.partial(
    pl.kernel,
    out_type=[
        jax.ShapeDtypeStruct((D * B,), jnp.int32),
        jax.ShapeDtypeStruct((D * B,), jnp.float32),
    ],
    mesh=_mesh,
    compiler_params=pltpu.CompilerParams(needs_layout_passes=False),
    scratch_types=[
        pltpu.VMEM((K,), jnp.float32),
        pltpu.VMEM((QS,), jnp.float32),
        pltpu.VMEM((K,), jnp.int32),
        pltpu.VMEM((K,), jnp.float32),
        pltpu.VMEM((K,), jnp.int32),
        pltpu.VMEM((NBUCK,), jnp.int32),
        pltpu.VMEM((NBUCK + 16,), jnp.int32),
        pltpu.VMEM((NBUCK,), jnp.int32),
        pltpu.VMEM((NBUCK,), jnp.int32),
        pltpu.VMEM((NBUCK,), jnp.int32),
        pltpu.VMEM((QS,), jnp.int32),
        pltpu.VMEM((QS,), jnp.float32),
    ],
)
def _vq_sc(qh, eh, zh, zqh, ev, qv, bbv, svv, sxv, cntv, startv, basev,
           wlov, whiv, zv, zqv):
    cid = lax.axis_index("c")
    sid = lax.axis_index("s")
    wid = sid * 2 + cid
    d = wid % 4
    r = wid // 4
    lane = lax.broadcasted_iota(jnp.int32, (16,), 0)
    l15 = jnp.full((16,), 15, jnp.int32)
    l0 = jnp.full((16,), 0, jnp.int32)

    @pl.when(d < D)
    def _():
        qoff = d * B + r * QS
        pltpu.sync_copy(eh.at[pl.ds(d * K, K)], ev)
        pltpu.sync_copy(qh.at[pl.ds(qoff, QS)], qv)

        def mm_body(i, c):
            mn, mx = c
            v = ev[pl.ds(i * 16, 16)]
            return jnp.minimum(mn, v), jnp.maximum(mx, v)

        mn, mx = lax.fori_loop(
            0, NKV, mm_body,
            (jnp.full((16,), jnp.inf, jnp.float32),
             jnp.full((16,), -jnp.inf, jnp.float32)),
        )
        mnv = jnp.broadcast_to(jnp.min(mn), (16,))
        rngv = jnp.broadcast_to(jnp.max(mx), (16,)) - mnv
        scv = jnp.where(rngv > 0.0, (NBUCK - 1.0) / rngv, 0.0)

        def zero_body(i, _):
            cntv[pl.ds(i * 16, 16)] = jnp.zeros((16,), jnp.int32)
            return 0

        lax.fori_loop(0, NBV, zero_body, 0)

        def hist_body(i, _):
            v = ev[pl.ds(i * 16, 16)]
            b = jnp.clip((v - mnv) * scv, 0.0, NBUCK - 1.0).astype(jnp.int32)
            bbv[pl.ds(i * 16, 16)] = b
            rc, is_last = plsc.scan_count(b)
            plsc.addupdate_scatter(cntv, [b], rc, mask=is_last)
            return 0

        lax.fori_loop(0, NKV, hist_body, 0)

        def pref_body(i, carry):
            c = cntv[pl.ds(i * 16, 16)]
            s = plsc.cumsum(c)
            excl = (s - c) + carry
            startv[pl.ds(i * 16, 16)] = excl
            basev[pl.ds(i * 16, 16)] = excl
            return carry + s[l15]

        carry = lax.fori_loop(0, NBV, pref_body, jnp.zeros((16,), jnp.int32))
        startv[pl.ds(NBUCK, 16)] = carry

        def scat_body(i, _):
            b = bbv[pl.ds(i * 16, 16)]
            v = ev[pl.ds(i * 16, 16)]
            rc, is_last = plsc.scan_count(b)
            slot = plsc.load_gather(basev, [b]) + (rc - 1)
            plsc.store_scatter(svv, [slot], v)
            plsc.store_scatter(sxv, [slot], lane + i * 16)
            plsc.addupdate_scatter(basev, [b], rc, mask=is_last)
            return 0

        lax.fori_loop(0, NKV, scat_body, 0)

        def fwd_body(i, carry):
            c = cntv[pl.ds(i * 16, 16)]
            g = lane + i * 16
            cand = jnp.where(c > 0, g, -1)
            incl = plsc.cummax(cand)
            shifted = incl[jnp.maximum(lane - 1, 0)]
            prevne = jnp.maximum(carry, jnp.where(lane == 0, -1, shifted))
            wl = plsc.load_gather(startv, [jnp.where(prevne >= 0, prevne, g)])
            wlov[pl.ds(i * 16, 16)] = wl
            return jnp.maximum(carry, incl[l15])

        lax.fori_loop(0, NBV, fwd_body, jnp.full((16,), -1, jnp.int32))

        def bwd_body(j, carry):
            i = (NBV - 1) - j
            c = cntv[pl.ds(i * 16, 16)]
            g = lane + i * 16
            cand = jnp.where(c > 0, g, NBUCK)
            suf = -lax.rev(plsc.cummax(lax.rev(-cand, (0,))), (0,))
            shifted = suf[jnp.minimum(lane + 1, 15)]
            nextne = jnp.minimum(carry, jnp.where(lane == 15, NBUCK, shifted))
            wh = plsc.load_gather(
                startv, [jnp.where(nextne < NBUCK, nextne, g) + 1])
            whiv[pl.ds(i * 16, 16)] = wh
            return jnp.minimum(carry, suf[l0])

        lax.fori_loop(0, NBV, bwd_body, jnp.full((16,), NBUCK, jnp.int32))

        def q_body(qi, _):
            q = qv[pl.ds(qi * 16, 16)]
            g = jnp.clip((q - mnv) * scv, 0.0, NBUCK - 1.0).astype(jnp.int32)
            wl = plsc.load_gather(wlov, [g])
            wh = plsc.load_gather(whiv, [g])

            def s_body(t, st):
                best, bidx, bval = st
                pos = wl + t
                m = pos < wh
                posc = jnp.where(m, pos, 0)
                sv = plsc.load_gather(svv, [posc])
                sx = jnp.where(m, plsc.load_gather(sxv, [posc]), K)
                diff = q - sv
                d2 = jnp.where(m, diff * diff, jnp.inf)
                better = (d2 < best) | ((d2 == best) & (sx < bidx))
                return (jnp.where(better, d2, best),
                        jnp.where(better, sx, bidx),
                        jnp.where(better, sv, bval))

            best, bidx, bval = lax.fori_loop(
                0, jnp.max(wh - wl), s_body,
                (jnp.full((16,), jnp.inf, jnp.float32),
                 jnp.full((16,), K, jnp.int32),
                 jnp.zeros((16,), jnp.float32)),
            )
            zv[pl.ds(qi * 16, 16)] = bidx
            zqv[pl.ds(qi * 16, 16)] = bval
            return 0

        lax.fori_loop(0, NQV, q_body, 0)

        pltpu.sync_copy(zv, zh.at[pl.ds(qoff, QS)])
        pltpu.sync_copy(zqv, zqh.at[pl.ds(qoff, QS)])


def kernel(ze, e):
    qflat = ze.reshape(B, D).T.reshape(D * B)
    eflat = e.T.reshape(D * K)
    zf, zqf = _vq_sc(qflat, eflat)
    z = zf.reshape(D, B).T
    zq = zqf.reshape(D, B).T
    return (z, zq)
